# baseline (speedup 1.0000x reference)
import functools

import jax
import jax.numpy as jnp
from jax import lax
from jax.experimental import pallas as pl
from jax.experimental.pallas import tpu as pltpu
from jax.experimental.pallas import tpu_sc as plsc

_D = 1024
_DH = _D // 2
_B, _S = 4, 8192
_NTOK = _B * _S
_ROWS = 2 * _NTOK
_NC, _NS = 2, 16
_NW = _NC * _NS
_TPW = _NTOK // _NW
_CT = 16
_NBUF = 4
_D_INFLIGHT = _NBUF // 2
_NITER = _TPW // _CT
_NL = 16


def _prep_body(gid_tab, gty_tab, row_tab, col_tab, iw, pw, cw,
               gids, gtys, rpos, h1, chi, colw, i1, g2t):
    c = cw[...]
    e = jnp.exp(c - jnp.max(c))
    w = e / jnp.sum(e)
    w0 = w[0, 0]
    w1 = w[0, 1]
    a = w0 * iw[0, 0]
    b = w0 * pw[0, 0]

    def _l2n(x):
        n = jnp.sqrt(jnp.sum(x * x, axis=-1, keepdims=True))
        return x / jnp.maximum(n, 1e-12)

    gin = a * _l2n(gid_tab[...])
    gtn = b * _l2n(gty_tab[...])
    row = w1 * row_tab[...]
    col = w1 * col_tab[...]

    for g in range(10):
        for y in range(2):
            k = g * 2 + y
            glo = (gin[g, :_DH] + gtn[y, :_DH])[None, :]
            h1[pl.ds(k * 30, 30)] = glo + row
            chi[pl.ds(k, 1)] = (gin[g, _DH:] + gtn[y, _DH:])[None, :]
    colw[...] = col

    gt = gids[...] * 2 + gtys[...]
    g2t[...] = gt
    i1[...] = gt * 30 + rpos[...]


_prep = pl.pallas_call(
    _prep_body,
    out_shape=[
        jax.ShapeDtypeStruct((600, _DH), jnp.float32),
        jax.ShapeDtypeStruct((20, _DH), jnp.float32),
        jax.ShapeDtypeStruct((30, _DH), jnp.float32),
        jax.ShapeDtypeStruct((_B, _S), jnp.int32),
        jax.ShapeDtypeStruct((_B, _S), jnp.int32),
    ],
)


def _sc_body(h1_hbm, chi_hbm, colw_hbm, i1_hbm, g2t_hbm, c_hbm, out_hbm,
             chi_v, colw_v, i1_all, g2t_all, c_all, *scr):
    wid = lax.axis_index("s") * _NC + lax.axis_index("c")
    tbase = wid * _TPW

    rows = scr[:_NBUF]
    sg = scr[_NBUF:2 * _NBUF]
    so = scr[2 * _NBUF:]

    brow = wid // (_S // _TPW)
    cbase = (wid % (_S // _TPW)) * _TPW
    pltpu.sync_copy(chi_hbm, chi_v)
    pltpu.sync_copy(colw_hbm, colw_v)
    pltpu.sync_copy(i1_hbm.at[brow, pl.ds(cbase, _TPW)],
                    i1_all.at[pl.ds(0, _TPW)])
    pltpu.sync_copy(g2t_hbm.at[brow, pl.ds(cbase, _TPW)],
                    g2t_all.at[pl.ds(0, _TPW)])
    pltpu.sync_copy(c_hbm.at[brow, pl.ds(cbase, _TPW)],
                    c_all.at[pl.ds(0, _TPW)])

    def _g1(i, buf):
        return (h1_hbm.at[i1_all.at[pl.ds(i * _CT, _CT)]],
                buf.at[:, pl.ds(0, _DH)])

    def _dst(i):
        tok = pl.multiple_of(tbase + i * _CT, 8)
        return out_hbm.at[pl.ds(tok, _CT)]

    def _gstart(i, b):
        pltpu.async_copy(*_g1(i, rows[b]), sg[b])

    def _gwait(i, b):
        pltpu.make_async_copy(*_g1(i, rows[b]), sg[b]).wait()

    def _wstart(i, b):
        pltpu.async_copy(rows[b], _dst(i), so[b])

    def _wwait(i, b):
        pltpu.make_async_copy(rows[b], _dst(i), so[b]).wait()

    def _fill(i, b):
        buf = rows[b]

        def tok(t, carry):
            g = g2t_all[pl.ds(i * _CT + t, _NL)][0]
            cc = c_all[pl.ds(i * _CT + t, _NL)][0]

            def col(v, carry2):
                off = pl.multiple_of(v * _NL, _NL)
                buf[t, pl.ds(_DH + off, _NL)] = (
                    chi_v[g, pl.ds(off, _NL)] + colw_v[cc, pl.ds(off, _NL)])
                return carry2

            return lax.fori_loop(0, _DH // _NL, col, carry, unroll=8)

        lax.fori_loop(0, _CT, tok, 0)

    dd = _D_INFLIGHT
    nj = _NITER // _NBUF
    for b in range(dd):
        _gstart(b, b)

    def step(j, carry):
        for k in range(_NBUF):
            i = _NBUF * j + k
            _fill(i, k)
            _gwait(i, k)
            _wstart(i, k)
            bn = (k + dd) % _NBUF
            if k < _NBUF - dd:
                @pl.when(j > 0)
                def _():
                    _wwait(i - (_NBUF - dd), bn)
                _gstart(i + dd, bn)
            else:
                _wwait(i - (_NBUF - dd), bn)

                @pl.when(j < nj - 1)
                def _():
                    _gstart(i + dd, bn)
        return carry

    lax.fori_loop(0, nj, step, 0)
    for c in range(_NITER - dd, _NITER):
        _wwait(c, c % _NBUF)


@functools.cache
def _sc_gather():
    return pl.kernel(
        _sc_body,
        out_type=jax.ShapeDtypeStruct((_NTOK, _D), jnp.float32),
        mesh=plsc.VectorSubcoreMesh(core_axis_name="c", subcore_axis_name="s",
                                    num_cores=_NC, num_subcores=_NS),
        scratch_types=(
            [pltpu.VMEM((20, _DH), jnp.float32),
             pltpu.VMEM((30, _DH), jnp.float32)]
            + [pltpu.VMEM((_TPW + _NL,), jnp.int32)] * 3
            + [pltpu.VMEM((_CT, _D), jnp.float32)] * _NBUF
            + [pltpu.SemaphoreType.DMA] * (2 * _NBUF)
        ),
    )


def kernel(grid_ids, grid_types, row_positions, col_positions,
           grid_id_table, grid_type_table, row_table, col_table,
           input_weight, position_weight, combine_weights):
    gids = grid_ids.astype(jnp.int32)
    gtys = grid_types.astype(jnp.int32)
    rpos = row_positions.astype(jnp.int32)
    cpos = col_positions.astype(jnp.int32)

    h1, chi, colw, i1, g2t = _prep(
        grid_id_table, grid_type_table, row_table, col_table,
        input_weight.reshape(1, 1), position_weight.reshape(1, 1),
        combine_weights.reshape(1, 4), gids, gtys, rpos)

    out = _sc_gather()(h1, chi, colw, i1, g2t, cpos)
    return out.reshape(_B, _S, _D)


# speedup vs baseline: 1.6858x; 1.6858x over previous
import functools

import jax
import jax.numpy as jnp
from jax import lax
from jax.experimental import pallas as pl
from jax.experimental.pallas import tpu as pltpu
from jax.experimental.pallas import tpu_sc as plsc

_D = 1024
_DH = _D // 2
_B, _S = 4, 8192
_NTOK = _B * _S
_ROWS = 2 * _NTOK
_NC, _NS = 2, 16
_NW = _NC * _NS
_TPW = _NTOK // _NW
_CT = 8
_NBUF = 8
_D_INFLIGHT = _NBUF // 2
_NITER = _TPW // _CT


def _prep_body(gid_tab, gty_tab, row_tab, col_tab, iw, pw, cw,
               gids, gtys, rpos, cpos, h, i1, i2):
    c = cw[...]
    e = jnp.exp(c - jnp.max(c))
    w = e / jnp.sum(e)
    w0 = w[0, 0]
    w1 = w[0, 1]
    a = w0 * iw[0, 0]
    b = w0 * pw[0, 0]

    def _l2n(x):
        n = jnp.sqrt(jnp.sum(x * x, axis=-1, keepdims=True))
        return x / jnp.maximum(n, 1e-12)

    gin = a * _l2n(gid_tab[...])
    gtn = b * _l2n(gty_tab[...])
    row = w1 * row_tab[...]
    col = w1 * col_tab[...]

    for g in range(10):
        for y in range(2):
            base = (g * 2 + y) * 30
            glo = (gin[g, :_DH] + gtn[y, :_DH])[None, :]
            ghi = (gin[g, _DH:] + gtn[y, _DH:])[None, :]
            h[pl.ds(base, 30)] = glo + row
            h[pl.ds(600 + base, 30)] = ghi + col

    base = (gids[...] * 2 + gtys[...]) * 30
    i1[...] = base + rpos[...]
    i2[...] = 600 + base + cpos[...]


_prep = pl.pallas_call(
    _prep_body,
    out_shape=[
        jax.ShapeDtypeStruct((1200, _DH), jnp.float32),
        jax.ShapeDtypeStruct((_B, _S), jnp.int32),
        jax.ShapeDtypeStruct((_B, _S), jnp.int32),
    ],
)


def _sc_body(h_hbm, idx1_hbm, idx2_hbm, out_hbm, idx1_all, idx2_all, *scr):
    wid = lax.axis_index("s") * _NC + lax.axis_index("c")
    tbase = wid * _TPW

    rows = scr[:_NBUF]
    sg = scr[_NBUF:2 * _NBUF]
    so = scr[2 * _NBUF:]

    brow = wid // (_S // _TPW)
    cbase = (wid % (_S // _TPW)) * _TPW
    pltpu.sync_copy(idx1_hbm.at[brow, pl.ds(cbase, _TPW)], idx1_all)
    pltpu.sync_copy(idx2_hbm.at[brow, pl.ds(cbase, _TPW)], idx2_all)

    def _g1(i, buf):
        return (h_hbm.at[idx1_all.at[pl.ds(i * _CT, _CT)]],
                buf.at[:, pl.ds(0, _DH)])

    def _g2(i, buf):
        return (h_hbm.at[idx2_all.at[pl.ds(i * _CT, _CT)]],
                buf.at[:, pl.ds(_DH, _DH)])

    def _dst(i):
        tok = pl.multiple_of(tbase + i * _CT, 8)
        return out_hbm.at[pl.ds(tok, _CT)]

    def _gstart(i, b):
        pltpu.async_copy(*_g1(i, rows[b]), sg[b])
        pltpu.async_copy(*_g2(i, rows[b]), sg[b])

    def _gwait(i, b):
        pltpu.make_async_copy(*_g1(i, rows[b]), sg[b]).wait()
        pltpu.make_async_copy(*_g2(i, rows[b]), sg[b]).wait()

    def _wstart(i, b):
        pltpu.async_copy(rows[b], _dst(i), so[b])

    def _wwait(i, b):
        pltpu.make_async_copy(rows[b], _dst(i), so[b]).wait()

    dd = _D_INFLIGHT
    nj = _NITER // _NBUF
    for b in range(dd):
        _gstart(b, b)

    def step(j, carry):
        for k in range(_NBUF):
            i = _NBUF * j + k
            _gwait(i, k)
            _wstart(i, k)
            bn = (k + dd) % _NBUF
            if k < _NBUF - dd:
                @pl.when(j > 0)
                def _():
                    _wwait(i - (_NBUF - dd), bn)
                _gstart(i + dd, bn)
            else:
                _wwait(i - (_NBUF - dd), bn)

                @pl.when(j < nj - 1)
                def _():
                    _gstart(i + dd, bn)
        return carry

    lax.fori_loop(0, nj, step, 0)
    for c in range(_NITER - dd, _NITER):
        _wwait(c, c % _NBUF)


@functools.cache
def _sc_gather():
    return pl.kernel(
        _sc_body,
        out_type=jax.ShapeDtypeStruct((_NTOK, _D), jnp.float32),
        mesh=plsc.VectorSubcoreMesh(core_axis_name="c", subcore_axis_name="s",
                                    num_cores=_NC, num_subcores=_NS),
        scratch_types=(
            [pltpu.VMEM((_TPW,), jnp.int32)] * 2
            + [pltpu.VMEM((_CT, _D), jnp.float32)] * _NBUF
            + [pltpu.SemaphoreType.DMA] * (2 * _NBUF)
        ),
    )


def kernel(grid_ids, grid_types, row_positions, col_positions,
           grid_id_table, grid_type_table, row_table, col_table,
           input_weight, position_weight, combine_weights):
    gids = grid_ids.astype(jnp.int32)
    gtys = grid_types.astype(jnp.int32)
    rpos = row_positions.astype(jnp.int32)
    cpos = col_positions.astype(jnp.int32)

    htab, i1, i2 = _prep(
        grid_id_table, grid_type_table, row_table, col_table,
        input_weight.reshape(1, 1), position_weight.reshape(1, 1),
        combine_weights.reshape(1, 4), gids, gtys, rpos, cpos)

    out = _sc_gather()(htab, i1, i2)
    return out.reshape(_B, _S, _D)
